# baseline (speedup 1.0000x reference)
import functools
import math

import numpy as np
import jax
import jax.numpy as jnp
from jax import lax
from jax.experimental import pallas as pl
from jax.experimental.pallas import tpu as pltpu

_SPARSITY = 0.25
_SLOPE1 = 10.0
_SLOPE2 = 10.0
_BM = 8
_RADIX_STEPS = 12


def _mask_kernel(logits_ref, logits8_ref, sample8_ref,
                 pm_ref, out_ref,
                 pm8_ref, bin8_ref, tile_ref, sem, sem_pm,
                 *, M, N, k_low, k_high, w_low, w_high):
    C = N // 8

    probs_r = jax.nn.sigmoid(_SLOPE1 * logits_ref[:])
    x_bar = jnp.sum(probs_r) / N

    r = _SPARSITY / x_bar
    beta = (1.0 - _SPARSITY) / (1.0 - x_bar)
    le = (r <= 1.0).astype(jnp.float32)
    probs8 = jax.nn.sigmoid(_SLOPE1 * logits8_ref[:])
    pm8 = le * probs8 * r + (1.0 - le) * (1.0 - (1.0 - probs8) * beta)
    pm8_ref[:] = pm8
    inter8 = jax.nn.sigmoid(_SLOPE2 * (pm8 - sample8_ref[:]))

    for j in range(8):
        pltpu.make_async_copy(
            pm8_ref.at[j], pm_ref.at[0, 0, pl.ds(j * C, C)], sem_pm).start()

    bits = lax.bitcast_convert_type(inter8, jnp.int32)
    target = k_high + 1
    lo = jnp.int32(0)
    hi = jnp.int32(0x3F800000)
    for _ in range(_RADIX_STEPS):
        d = jnp.maximum((hi - lo) // 8, 1)
        ts = [jnp.minimum(lo + j * d, hi) for j in range(1, 8)]
        cs = [jnp.sum((bits <= t).astype(jnp.int32)) for t in ts]
        ge = [(c >= target).astype(jnp.int32) for c in cs]
        jmin = 8 - (ge[0] + ge[1] + ge[2] + ge[3] + ge[4] + ge[5] + ge[6])
        bounds = [lo] + ts + [hi]
        new_lo, new_hi = bounds[0], bounds[1]
        for j in range(1, 8):
            sel = jmin == (j + 1)
            new_lo = jnp.where(sel, bounds[j], new_lo)
            new_hi = jnp.where(sel, bounds[j + 1], new_hi)
        lo, hi = new_lo, new_hi

    v_high = lax.bitcast_convert_type(hi, jnp.float32)
    if k_high == k_low:
        v_low = v_high
    else:
        pred = hi - 1
        c_pred = jnp.sum((bits <= pred).astype(jnp.int32))
        below_max = jnp.max(jnp.where(bits <= pred, bits, 0))
        lo_bits = jnp.where(c_pred >= (k_low + 1), below_max, hi)
        v_low = lax.bitcast_convert_type(lo_bits, jnp.float32)

    thresh = v_low * w_low + v_high * w_high

    bin8_ref[:] = (inter8 >= thresh).astype(jnp.float32)

    for j in range(8):
        pltpu.make_async_copy(
            bin8_ref.at[j], out_ref.at[0, 0, pl.ds(j * C, C)], sem).start()
    for j in range(8):
        pltpu.make_async_copy(
            bin8_ref.at[j], out_ref.at[0, 0, pl.ds(j * C, C)], sem).wait()

    for t in range(_BM):
        pltpu.make_async_copy(
            out_ref.at[0, 0, :], tile_ref.at[t], sem).start()
    for t in range(_BM):
        pltpu.make_async_copy(
            out_ref.at[0, 0, :], tile_ref.at[t], sem).wait()

    n_tiles = M // _BM

    def start_dma(i, _):
        pltpu.make_async_copy(
            tile_ref, out_ref.at[0, pl.ds(i * _BM, _BM), :], sem).start()
        return 0

    lax.fori_loop(0, n_tiles, start_dma, 0)

    def wait_dma(i, _):
        pltpu.make_async_copy(
            tile_ref, out_ref.at[0, pl.ds(0, _BM), :], sem).wait()
        return 0

    lax.fori_loop(0, n_tiles, wait_dma, 0)

    for j in range(8):
        pltpu.make_async_copy(
            pm8_ref.at[j], pm_ref.at[0, 0, pl.ds(j * C, C)], sem_pm).wait()


def kernel(logits, sample_mask):
    B, one, N = logits.shape
    M = 2048
    n_total = logits.size
    q = np.float32(1.0 - _SPARSITY)
    idx = np.float32(q * np.float32(n_total - 1))
    low = np.floor(idx)
    w_high = np.float32(idx - low)
    w_low = np.float32(np.float32(1.0) - w_high)
    k_low = int(low)
    k_high = int(math.ceil(float(idx)))

    kern = functools.partial(_mask_kernel, M=M, N=N, k_low=k_low,
                             k_high=k_high, w_low=w_low, w_high=w_high)
    logits8 = logits.reshape(8, n_total // 8)
    sample8 = sample_mask.reshape(8, n_total // 8)
    pm, final = pl.pallas_call(
        kern,
        in_specs=[
            pl.BlockSpec(memory_space=pltpu.MemorySpace.VMEM),
            pl.BlockSpec(memory_space=pltpu.MemorySpace.VMEM),
            pl.BlockSpec(memory_space=pltpu.MemorySpace.VMEM),
        ],
        out_specs=[
            pl.BlockSpec(memory_space=pl.ANY),
            pl.BlockSpec(memory_space=pl.ANY),
        ],
        out_shape=[
            jax.ShapeDtypeStruct((B, 1, N), jnp.float32),
            jax.ShapeDtypeStruct((B, M, N), jnp.float32),
        ],
        scratch_shapes=[
            pltpu.VMEM((8, N // 8), jnp.float32),
            pltpu.VMEM((8, N // 8), jnp.float32),
            pltpu.VMEM((_BM, N), jnp.float32),
            pltpu.SemaphoreType.DMA,
            pltpu.SemaphoreType.DMA,
        ],
    )(logits, logits8, sample8)
    return (pm, final)
